# Initial kernel scaffold; baseline (speedup 1.0000x reference)
import jax
import jax.numpy as jnp
from jax.experimental import pallas as pl


def kernel(x, edge_index, edge_val, W):
    raise NotImplementedError("write your pallas kernel here")


# speedup vs baseline: 10.1572x; 10.1572x over previous
import functools

import jax
import jax.numpy as jnp
from jax import lax
from jax.experimental import pallas as pl
from jax.experimental.pallas import tpu as pltpu
from jax.experimental.pallas import tpu_sc as plsc

N_NODES = 10000
D = 128
SUPPORT = 2
N_EDGES = 320000

NSUB = 16
CHUNK = 128
CHUNKS_PER_TILE = 160
EB = 32
NBLK = CHUNKS_PER_TILE // EB
EDGES_PER_TILE = CHUNK * CHUNKS_PER_TILE
E_PAD = EDGES_PER_TILE * NSUB
ROWS_PER_REL = E_PAD // CHUNK
SLAB = 624
TAIL = N_NODES - SLAB * NSUB


def _sc_spmm(x, rows2, cols2, vals2):
    mesh = plsc.VectorSubcoreMesh(core_axis_name="c", subcore_axis_name="s")

    @functools.partial(
        pl.kernel,
        out_type=jax.ShapeDtypeStruct((SUPPORT, N_NODES, D), jnp.float32),
        mesh=mesh,
        scratch_types=[
            pltpu.VMEM((EB, CHUNK), jnp.int32),
            pltpu.VMEM((EB, CHUNK), jnp.int32),
            pltpu.VMEM((EB, CHUNK), jnp.float32),
            pltpu.VMEM((CHUNK, D), jnp.float32),
            pltpu.VMEM((CHUNK, D), jnp.float32),
            pltpu.VMEM_SHARED((N_NODES, D), jnp.float32),
            pltpu.SemaphoreType.DMA,
            pltpu.SemaphoreType.DMA,
            pltpu.SemaphoreType.DMA,
            pltpu.SemaphoreType.DMA,
            pltpu.SemaphoreType.DMA,
        ],
    )
    def sc_kernel(x_hbm, rows_hbm, cols_hbm, vals_hbm, out_hbm,
                  cols_v, rows_v, vals_v, m0, m1, acc_sh,
                  gs0, gs1, ss0, ss1, es):
        ci = lax.axis_index("c")
        sid = lax.axis_index("s")
        row0 = ci * ROWS_PER_REL + sid * CHUNKS_PER_TILE

        zero16 = jnp.zeros((16,), jnp.float32)

        def zrow(r, _):
            for q in range(D // 16):
                m0[r, pl.ds(q * 16, 16)] = zero16
            return 0

        lax.fori_loop(0, CHUNK, zrow, 0)

        slab0 = sid * SLAB
        for b in range(SLAB // CHUNK):
            pltpu.sync_copy(m0, acc_sh.at[pl.ds(slab0 + b * CHUNK, CHUNK)])
        rem = SLAB - (SLAB // CHUNK) * CHUNK
        pltpu.sync_copy(m0.at[pl.ds(0, rem)],
                        acc_sh.at[pl.ds(slab0 + (SLAB // CHUNK) * CHUNK, rem)])

        @pl.when(sid == NSUB - 1)
        def _():
            pltpu.sync_copy(m0.at[pl.ds(0, TAIL)],
                            acc_sh.at[pl.ds(NSUB * SLAB, TAIL)])

        plsc.subcore_barrier()

        def scale(buf, g):
            def row_body(b, _):
                vv = vals_v[g, pl.ds(b * 16, 16)]
                for r2 in range(16):
                    bv = vv.at[jnp.full((16,), r2, jnp.int32)].get(
                        mode="promise_in_bounds")
                    r = b * 16 + r2
                    for q in range(D // 16):
                        sl = pl.ds(q * 16, 16)
                        buf[r, sl] = buf[r, sl] * bv
                return 0

            lax.fori_loop(0, CHUNK // 16, row_body, 0)

        def gather(g, buf, sem):
            return pltpu.async_copy(x_hbm.at[cols_v.at[g]], buf, sem)

        def scatter(buf, g, sem):
            return pltpu.async_copy(buf, acc_sh.at[rows_v.at[g]], sem,
                                    add=True)

        def wait_gather(buf, sem):
            pltpu.make_async_copy(x_hbm.at[cols_v.at[0]], buf, sem).wait()

        def wait_scatter(buf, sem):
            pltpu.make_async_copy(buf, acc_sh.at[rows_v.at[0]], sem).wait()

        def blk_body(blk, _):
            base_row = row0 + blk * EB
            pltpu.sync_copy(cols_hbm.at[pl.ds(base_row, EB)], cols_v)
            pltpu.sync_copy(rows_hbm.at[pl.ds(base_row, EB)], rows_v)
            pltpu.sync_copy(vals_hbm.at[pl.ds(base_row, EB)], vals_v)

            gather(0, m0, gs0)

            def body(i, _):
                g0 = 2 * i
                g1 = 2 * i + 1

                @pl.when(i > 0)
                def _():
                    wait_scatter(m1, ss1)

                gather(g1, m1, gs1)
                wait_gather(m0, gs0)
                scale(m0, g0)
                scatter(m0, g0, ss0)
                wait_gather(m1, gs1)
                scale(m1, g1)
                wait_scatter(m0, ss0)

                @pl.when(i < EB // 2 - 1)
                def _():
                    gather(g0 + 2, m0, gs0)

                scatter(m1, g1, ss1)
                return 0

            lax.fori_loop(0, EB // 2, body, 0)
            wait_scatter(m1, ss1)
            return 0

        lax.fori_loop(0, NBLK, blk_body, 0)

        plsc.subcore_barrier()

        pltpu.sync_copy(acc_sh.at[pl.ds(slab0, SLAB)],
                        out_hbm.at[ci, pl.ds(slab0, SLAB)])

        @pl.when(sid == NSUB - 1)
        def _():
            pltpu.sync_copy(acc_sh.at[pl.ds(NSUB * SLAB, TAIL)],
                            out_hbm.at[ci, pl.ds(NSUB * SLAB, TAIL)])

    return sc_kernel(x, rows2, cols2, vals2)


def _mm_body(s_ref, w_ref, o_ref):
    a = jnp.dot(s_ref[0], w_ref[0:D, :], preferred_element_type=jnp.float32)
    b = jnp.dot(s_ref[1], w_ref[D:2 * D, :], preferred_element_type=jnp.float32)
    o_ref[...] = jnp.maximum(a + b, 0.0)


def _mm_relu(supports, W):
    BM = 2000
    grid = (N_NODES // BM,)
    return pl.pallas_call(
        _mm_body,
        grid=grid,
        in_specs=[
            pl.BlockSpec((SUPPORT, BM, D), lambda i: (0, i, 0)),
            pl.BlockSpec((SUPPORT * D, D), lambda i: (0, 0)),
        ],
        out_specs=pl.BlockSpec((BM, D), lambda i: (i, 0)),
        out_shape=jax.ShapeDtypeStruct((N_NODES, D), jnp.float32),
    )(supports, W)


def kernel(x, edge_index, edge_val, W):
    ei = edge_index.astype(jnp.int32)
    npad = E_PAD - N_EDGES
    pad_i = jnp.broadcast_to(
        jnp.arange(npad, dtype=jnp.int32) % N_NODES, (SUPPORT, npad))
    pad_f = jnp.zeros((SUPPORT, npad), jnp.float32)
    rows2 = jnp.concatenate([ei[:, 0, :], pad_i], axis=1).reshape(-1, CHUNK)
    cols2 = jnp.concatenate([ei[:, 1, :], pad_i], axis=1).reshape(-1, CHUNK)
    vals2 = jnp.concatenate([edge_val, pad_f], axis=1).reshape(-1, CHUNK)
    supports = _sc_spmm(x, rows2, cols2, vals2)
    return _mm_relu(supports, W)
